# Initial kernel scaffold; baseline (speedup 1.0000x reference)
import jax
import jax.numpy as jnp
from jax.experimental import pallas as pl


def kernel(x, edge_index, W_self1, W_neigh1, b1, W_self2, W_neigh2, b2, W_self3, W_neigh3, b3):
    raise NotImplementedError("write your pallas kernel here")


# speedup vs baseline: 5.5594x; 5.5594x over previous
import functools

import jax
import jax.numpy as jnp
from jax import lax
from jax.experimental import pallas as pl
from jax.experimental.pallas import tpu as pltpu
from jax.experimental.pallas import tpu_sc as plsc

N_NODES = 10000
F = 128
E = 320000
NC = 2
NS = 16
NW = NC * NS
EPW = E // NW
CHUNK = 80
NCHUNK = EPW // CHUNK
NP = 10240
RPT = NP // NS

_MESH = plsc.VectorSubcoreMesh(core_axis_name="c", subcore_axis_name="s")


def _make_agg(with_deg):
    out_type = [jax.ShapeDtypeStruct((NC * NP, F), jnp.float32)]
    scratch = [
        pltpu.VMEM((CHUNK,), jnp.int32),
        pltpu.VMEM((CHUNK,), jnp.int32),
        pltpu.VMEM((CHUNK, F), jnp.float32),
        pltpu.VMEM_SHARED((NP, F), jnp.float32),
        pltpu.SemaphoreType.DMA,
    ]
    if with_deg:
        out_type.append(jax.ShapeDtypeStruct((NC * NP,), jnp.float32))
        scratch += [
            pltpu.VMEM((CHUNK,), jnp.float32),
            pltpu.VMEM_SHARED((NP,), jnp.float32),
        ]

    @functools.partial(
        pl.kernel, mesh=_MESH,
        out_type=out_type,
        scratch_types=scratch,
    )
    def agg(*refs):
        if with_deg:
            (p_hbm, src_hbm, dst_hbm, zrow_hbm, ones_hbm, zdeg_hbm,
             out_hbm, deg_hbm, srcv, dstv, rows, accum, sem, onesv, dega) = refs
        else:
            (p_hbm, src_hbm, dst_hbm, zrow_hbm,
             out_hbm, srcv, dstv, rows, accum, sem) = refs
        c = lax.axis_index("c")
        s = lax.axis_index("s")
        wid = c * NS + s
        row0 = s * RPT
        pltpu.sync_copy(zrow_hbm, accum.at[pl.ds(row0, RPT)])
        if with_deg:
            pltpu.sync_copy(zdeg_hbm, dega.at[pl.ds(row0, RPT)])
            pltpu.sync_copy(ones_hbm, onesv)
        plsc.subcore_barrier()
        base = wid * EPW

        def step(i, carry):
            off = base + i * CHUNK
            pltpu.sync_copy(src_hbm.at[pl.ds(off, CHUNK)], srcv)
            pltpu.sync_copy(dst_hbm.at[pl.ds(off, CHUNK)], dstv)
            pltpu.async_copy(p_hbm.at[srcv], rows, sem).wait()
            pltpu.sync_copy(rows, accum.at[dstv], add=True)
            if with_deg:
                pltpu.sync_copy(onesv, dega.at[dstv], add=True)
            return carry

        lax.fori_loop(0, NCHUNK, step, 0)
        plsc.subcore_barrier()
        out_row0 = c * NP + row0
        pltpu.sync_copy(accum.at[pl.ds(row0, RPT)],
                        out_hbm.at[pl.ds(out_row0, RPT)])
        if with_deg:
            pltpu.sync_copy(dega.at[pl.ds(row0, RPT)],
                            deg_hbm.at[pl.ds(out_row0, RPT)])

    return agg


@functools.partial(
    pl.kernel, mesh=_MESH,
    out_type=[jax.ShapeDtypeStruct((NC * NP,), jnp.float32)],
    scratch_types=[
        pltpu.VMEM((CHUNK,), jnp.int32),
        pltpu.VMEM((CHUNK,), jnp.int32),
        pltpu.VMEM((CHUNK,), jnp.float32),
        pltpu.VMEM_SHARED((NP,), jnp.float32),
        pltpu.SemaphoreType.DMA,
    ],
)
def _agg_scalar(q_hbm, src_hbm, dst_hbm, z_hbm, out_hbm,
                srcv, dstv, valv, accum, sem):
    c = lax.axis_index("c")
    s = lax.axis_index("s")
    wid = c * NS + s
    row0 = s * RPT
    pltpu.sync_copy(z_hbm, accum.at[pl.ds(row0, RPT)])
    plsc.subcore_barrier()
    base = wid * EPW

    def step(i, carry):
        off = base + i * CHUNK
        pltpu.sync_copy(src_hbm.at[pl.ds(off, CHUNK)], srcv)
        pltpu.sync_copy(dst_hbm.at[pl.ds(off, CHUNK)], dstv)
        pltpu.async_copy(q_hbm.at[srcv], valv, sem).wait()
        pltpu.sync_copy(valv, accum.at[dstv], add=True)
        return carry

    lax.fori_loop(0, NCHUNK, step, 0)
    plsc.subcore_barrier()
    pltpu.sync_copy(accum.at[pl.ds(row0, RPT)],
                    out_hbm.at[pl.ds(c * NP + row0, RPT)])


_agg_f_deg = _make_agg(True)
_agg_f = _make_agg(False)

_BR = 1000
_GRID = N_NODES // _BR
_QW = 8


def _row_spec(w):
    return pl.BlockSpec((_BR, w), lambda i: (i, 0))


def _full_spec(r, w):
    return pl.BlockSpec((r, w), lambda i: (0, 0))


def _inv_deg(d0, d1):
    return 1.0 / jnp.maximum(d0[...] + d1[...], 1.0)


def _tc1_body(x, p0, p1, d0, d1, ws, wn, b, o):
    n = (p0[...] + p1[...]) * _inv_deg(d0, d1)
    h = (jnp.dot(x[...], ws[...], preferred_element_type=jnp.float32)
         + jnp.dot(n, wn[...], preferred_element_type=jnp.float32)
         + b[0:1, :])
    o[...] = jnp.maximum(h, 0.0)


def _tc2_body(h1, p0, p1, d0, d1, ws, wn, b, wn3, oh, oq):
    n = (p0[...] + p1[...]) * _inv_deg(d0, d1)
    h2 = (jnp.dot(h1[...], ws[...], preferred_element_type=jnp.float32)
          + jnp.dot(n, wn[...], preferred_element_type=jnp.float32)
          + b[0:1, :])
    h2 = jnp.maximum(h2, 0.0)
    oh[...] = h2
    oq[...] = jnp.dot(h2, wn3[...], preferred_element_type=jnp.float32)


def _tc3_body(h2, q0, q1, d0, d1, ws, b, o):
    n = (q0[...] + q1[...]) * _inv_deg(d0, d1)
    o[...] = (jnp.dot(h2[...], ws[...], preferred_element_type=jnp.float32)
              + n + b[0:1, :])


def kernel(x, edge_index, W_self1, W_neigh1, b1, W_self2, W_neigh2, b2,
           W_self3, W_neigh3, b3):
    src = edge_index[0]
    dst = edge_index[1]
    zrow = jnp.zeros((RPT, F), jnp.float32)
    zsca = jnp.zeros((RPT,), jnp.float32)
    ones = jnp.ones((CHUNK,), jnp.float32)

    agg_x, deg = _agg_f_deg(x, src, dst, zrow, ones, zsca)
    p0, p1 = agg_x[:N_NODES], agg_x[NP:NP + N_NODES]
    d0 = deg[:N_NODES, None]
    d1 = deg[NP:NP + N_NODES, None]

    b1r = jnp.broadcast_to(b1[None, :], (8, F))
    h1 = pl.pallas_call(
        _tc1_body,
        grid=(_GRID,),
        in_specs=[_row_spec(F), _row_spec(F), _row_spec(F),
                  _row_spec(1), _row_spec(1),
                  _full_spec(F, F), _full_spec(F, F), _full_spec(8, F)],
        out_specs=_row_spec(F),
        out_shape=jax.ShapeDtypeStruct((N_NODES, F), jnp.float32),
    )(x, p0, p1, d0, d1, W_self1, W_neigh1, b1r)

    agg_h1 = _agg_f(h1, src, dst, zrow)[0]
    p0, p1 = agg_h1[:N_NODES], agg_h1[NP:NP + N_NODES]

    b2r = jnp.broadcast_to(b2[None, :], (8, F))
    wn3p = jnp.pad(W_neigh3, ((0, 0), (0, _QW - 1)))
    h2, q = pl.pallas_call(
        _tc2_body,
        grid=(_GRID,),
        in_specs=[_row_spec(F), _row_spec(F), _row_spec(F),
                  _row_spec(1), _row_spec(1),
                  _full_spec(F, F), _full_spec(F, F), _full_spec(8, F),
                  _full_spec(F, _QW)],
        out_specs=[_row_spec(F), _row_spec(_QW)],
        out_shape=[jax.ShapeDtypeStruct((N_NODES, F), jnp.float32),
                   jax.ShapeDtypeStruct((N_NODES, _QW), jnp.float32)],
    )(h1, p0, p1, d0, d1, W_self2, W_neigh2, b2r, wn3p)

    qflat = jnp.pad(q[:, 0], (0, NP - N_NODES))
    agg_q = _agg_scalar(qflat, src, dst, zsca)[0]
    q0 = agg_q[:N_NODES, None]
    q1 = agg_q[NP:NP + N_NODES, None]

    ws3p = jnp.pad(W_self3, ((0, 0), (0, _QW - 1)))
    b3r = jnp.broadcast_to(b3.reshape(1, 1), (8, _QW))
    out8 = pl.pallas_call(
        _tc3_body,
        grid=(_GRID,),
        in_specs=[_row_spec(F), _row_spec(1), _row_spec(1),
                  _row_spec(1), _row_spec(1),
                  _full_spec(F, _QW), _full_spec(8, _QW)],
        out_specs=_row_spec(_QW),
        out_shape=jax.ShapeDtypeStruct((N_NODES, _QW), jnp.float32),
    )(h2, q0, q1, d0, d1, ws3p, b3r)

    return out8[:, :1]


# speedup vs baseline: 11.5184x; 2.0719x over previous
import functools

import jax
import jax.numpy as jnp
from jax import lax
from jax.experimental import pallas as pl
from jax.experimental.pallas import tpu as pltpu
from jax.experimental.pallas import tpu_sc as plsc

N_NODES = 10000
F = 128
E = 320000
NC = 2
NS = 16
NW = NC * NS
CHUNK = 128
NCH = 80
EPW = NCH * CHUNK
EP = NW * EPW
NP = 10240
RPT = NP // NS

_MESH = plsc.VectorSubcoreMesh(core_axis_name="c", subcore_axis_name="s")


def _make_agg(scalar, with_deg):
    fshape = () if scalar else (F,)
    out_type = [jax.ShapeDtypeStruct((NC * NP,) + fshape, jnp.float32)]
    scratch = [
        pltpu.VMEM((2, CHUNK), jnp.int32),
        pltpu.VMEM((2, CHUNK), jnp.int32),
        pltpu.VMEM((CHUNK,) + fshape, jnp.float32),
        pltpu.VMEM((CHUNK,) + fshape, jnp.float32),
        pltpu.VMEM_SHARED((NP,) + fshape, jnp.float32),
        pltpu.SemaphoreType.DMA,
        pltpu.SemaphoreType.DMA,
        pltpu.SemaphoreType.DMA,
        pltpu.SemaphoreType.DMA,
    ]
    if with_deg:
        out_type.append(jax.ShapeDtypeStruct((NC * NP,), jnp.float32))
        scratch += [
            pltpu.VMEM((CHUNK,), jnp.float32),
            pltpu.VMEM_SHARED((NP,), jnp.float32),
        ]

    @functools.partial(
        pl.kernel, mesh=_MESH,
        out_type=out_type,
        scratch_types=scratch,
    )
    def agg(*refs):
        if with_deg:
            (p_hbm, e_hbm, z_hbm, ones_hbm, zsca_hbm,
             out_hbm, deg_hbm, e0, e1, r0, r1, accum,
             sg0, sg1, si0, si1, onesv, dega) = refs
        else:
            (p_hbm, e_hbm, z_hbm,
             out_hbm, e0, e1, r0, r1, accum, sg0, sg1, si0, si1) = refs
        c = lax.axis_index("c")
        s = lax.axis_index("s")
        wid = c * NS + s
        row0 = s * RPT
        pltpu.sync_copy(z_hbm, accum.at[pl.ds(row0, RPT)])
        if with_deg:
            pltpu.sync_copy(zsca_hbm, dega.at[pl.ds(row0, RPT)])
            pltpu.sync_copy(ones_hbm, onesv)
        plsc.subcore_barrier()

        bufs = ((e0, r0, sg0, si0), (e1, r1, sg1, si1))
        pltpu.async_copy(e_hbm.at[wid, 0], e0, si0)
        pltpu.async_copy(e_hbm.at[wid, 1], e1, si1)
        pltpu.make_async_copy(e_hbm.at[wid, 0], e0, si0).wait()
        pltpu.async_copy(p_hbm.at[e0.at[0]], r0, sg0)

        def step(g, carry):
            for b in range(2):
                eb, rb, sgb, sib = bufs[b]
                en, rn, sgn, sin = bufs[1 - b]
                i = 2 * g + b
                pltpu.make_async_copy(p_hbm.at[eb.at[0]], rb, sgb).wait()
                pltpu.make_async_copy(
                    e_hbm.at[wid, jnp.minimum(i + 1, NCH - 1)], en,
                    sin).wait()
                pltpu.async_copy(p_hbm.at[en.at[0]], rn, sgn)
                pltpu.sync_copy(rb, accum.at[eb.at[1]], add=True)
                if with_deg:
                    pltpu.sync_copy(onesv, dega.at[eb.at[1]], add=True)
                pltpu.async_copy(
                    e_hbm.at[wid, jnp.minimum(i + 2, NCH - 1)], eb, sib)
            return carry

        lax.fori_loop(0, NCH // 2, step, 0)
        pltpu.make_async_copy(p_hbm.at[e0.at[0]], r0, sg0).wait()
        pltpu.make_async_copy(e_hbm.at[wid, NCH - 1], e1, si1).wait()
        plsc.subcore_barrier()
        out_row0 = c * NP + row0
        pltpu.sync_copy(accum.at[pl.ds(row0, RPT)],
                        out_hbm.at[pl.ds(out_row0, RPT)])
        if with_deg:
            pltpu.sync_copy(dega.at[pl.ds(row0, RPT)],
                            deg_hbm.at[pl.ds(out_row0, RPT)])

    return agg


_agg_f_deg = _make_agg(False, True)
_agg_f = _make_agg(False, False)
_agg_scalar = _make_agg(True, False)

_BR = 1024
_GRID = NP // _BR
_QW = 8


def _row_spec(w):
    return pl.BlockSpec((_BR, w), lambda i: (i, 0))


def _full_spec(r, w):
    return pl.BlockSpec((r, w), lambda i: (0, 0))


def _inv_deg(d0, d1):
    return 1.0 / jnp.maximum(d0[...] + d1[...], 1.0)


def _tc1_body(x, p0, p1, d0, d1, ws, wn, b, o):
    n = (p0[...] + p1[...]) * _inv_deg(d0, d1)
    h = (jnp.dot(x[...], ws[...], preferred_element_type=jnp.float32)
         + jnp.dot(n, wn[...], preferred_element_type=jnp.float32)
         + b[0:1, :])
    o[...] = jnp.maximum(h, 0.0)


def _tc2_body(h1, p0, p1, d0, d1, ws, wn, b, wn3, oh, oq):
    n = (p0[...] + p1[...]) * _inv_deg(d0, d1)
    h2 = (jnp.dot(h1[...], ws[...], preferred_element_type=jnp.float32)
          + jnp.dot(n, wn[...], preferred_element_type=jnp.float32)
          + b[0:1, :])
    h2 = jnp.maximum(h2, 0.0)
    oh[...] = h2
    oq[...] = jnp.dot(h2, wn3[...], preferred_element_type=jnp.float32)


def _tc3_body(h2, q0, q1, d0, d1, ws, b, o):
    n = (q0[...] + q1[...]) * _inv_deg(d0, d1)
    o[...] = (jnp.dot(h2[...], ws[...], preferred_element_type=jnp.float32)
              + n + b[0:1, :])


def kernel(x, edge_index, W_self1, W_neigh1, b1, W_self2, W_neigh2, b2,
           W_self3, W_neigh3, b3):
    pad = (N_NODES + jnp.arange(EP - E, dtype=jnp.int32)
           % (NP - N_NODES)).astype(jnp.int32)
    src = jnp.concatenate([edge_index[0], pad]).reshape(NW, NCH, 1, CHUNK)
    dst = jnp.concatenate([edge_index[1], pad]).reshape(NW, NCH, 1, CHUNK)
    edges = jnp.concatenate([src, dst], axis=2)
    xp = jnp.pad(x, ((0, NP - N_NODES), (0, 0)))
    zrow = jnp.zeros((RPT, F), jnp.float32)
    zsca = jnp.zeros((RPT,), jnp.float32)
    ones = jnp.ones((CHUNK,), jnp.float32)

    agg_x, deg = _agg_f_deg(xp, edges, zrow, ones, zsca)
    p0, p1 = agg_x[:NP], agg_x[NP:]
    d0 = deg[:NP, None]
    d1 = deg[NP:, None]

    b1r = jnp.broadcast_to(b1[None, :], (8, F))
    h1 = pl.pallas_call(
        _tc1_body,
        grid=(_GRID,),
        in_specs=[_row_spec(F), _row_spec(F), _row_spec(F),
                  _row_spec(1), _row_spec(1),
                  _full_spec(F, F), _full_spec(F, F), _full_spec(8, F)],
        out_specs=_row_spec(F),
        out_shape=jax.ShapeDtypeStruct((NP, F), jnp.float32),
    )(xp, p0, p1, d0, d1, W_self1, W_neigh1, b1r)

    agg_h1 = _agg_f(h1, edges, zrow)[0]
    p0, p1 = agg_h1[:NP], agg_h1[NP:]

    b2r = jnp.broadcast_to(b2[None, :], (8, F))
    wn3p = jnp.pad(W_neigh3, ((0, 0), (0, _QW - 1)))
    h2, q = pl.pallas_call(
        _tc2_body,
        grid=(_GRID,),
        in_specs=[_row_spec(F), _row_spec(F), _row_spec(F),
                  _row_spec(1), _row_spec(1),
                  _full_spec(F, F), _full_spec(F, F), _full_spec(8, F),
                  _full_spec(F, _QW)],
        out_specs=[_row_spec(F), _row_spec(_QW)],
        out_shape=[jax.ShapeDtypeStruct((NP, F), jnp.float32),
                   jax.ShapeDtypeStruct((NP, _QW), jnp.float32)],
    )(h1, p0, p1, d0, d1, W_self2, W_neigh2, b2r, wn3p)

    agg_q = _agg_scalar(q[:, 0], edges, zsca)[0]
    q0 = agg_q[:NP, None]
    q1 = agg_q[NP:, None]

    ws3p = jnp.pad(W_self3, ((0, 0), (0, _QW - 1)))
    b3r = jnp.broadcast_to(b3.reshape(1, 1), (8, _QW))
    out8 = pl.pallas_call(
        _tc3_body,
        grid=(_GRID,),
        in_specs=[_row_spec(F), _row_spec(1), _row_spec(1),
                  _row_spec(1), _row_spec(1),
                  _full_spec(F, _QW), _full_spec(8, _QW)],
        out_specs=_row_spec(_QW),
        out_shape=jax.ShapeDtypeStruct((NP, _QW), jnp.float32),
    )(h2, q0, q1, d0, d1, ws3p, b3r)

    return out8[:N_NODES, :1]
